# baseline (speedup 1.0000x reference)
import functools

import jax
import jax.numpy as jnp
from jax.experimental import pallas as pl
from jax.experimental.pallas import tpu as pltpu

_BM = 1024
_BL = 1024

_T_DIMS = (((0,), (0,)), ((), ()))


def _split_hi_lo_rows(t):
    hi = t.astype(jnp.bfloat16)
    lo = (t - hi.astype(jnp.float32)).astype(jnp.bfloat16)
    return jnp.concatenate([hi, lo], axis=0)


def _sym_deg_kernel(adj_ij, adj_ji, x_ref, w1_ref, a_out, dinv_out, z1_out,
                    deg_r, deg_c, *, bm, n, ni, r):
    i = pl.program_id(0)
    j = pl.program_id(1)

    @pl.when((i == 0) & (j == 0))
    def _():
        deg_r[...] = jnp.zeros_like(deg_r)
        deg_c[...] = jnp.zeros_like(deg_c)

    def finish(mv):
        a_out[...] = mv.astype(jnp.int8)
        rs = jnp.sum(mv, axis=1, keepdims=True)
        deg_r[pl.ds(i * bm, bm), :] += rs

        @pl.when(j >= (i // r) * r + r)
        def _():
            cs = jnp.sum(mv, axis=0, keepdims=True)
            deg_c[:, pl.ds(j * bm, bm)] += cs

    @pl.when(j >= (i // r) * r)
    def _():
        a = adj_ij[...]
        at = adj_ji[...].T
        m = jnp.maximum(a, at)

        is_edge = ((i + 1) * bm > n) | ((j + 1) * bm > n)

        @pl.when(is_edge)
        def _():
            rid = jax.lax.broadcasted_iota(jnp.int32, (bm, 1), 0)
            cid = jax.lax.broadcasted_iota(jnp.int32, (1, bm), 1)
            valid = (rid < n - i * bm) & (cid < n - j * bm)
            finish(jnp.where(valid, m, 0.0))

        @pl.when(~is_edge)
        def _():
            finish(m)

    @pl.when((i == ni - 1) & (j == ni - 1))
    def _():
        deg_t = deg_r[...].T + deg_c[...]
        dinv_t = jnp.where(deg_t > 0.0,
                           jax.lax.rsqrt(jnp.maximum(deg_t, 1e-12)), 0.0)
        dinv_out[...] = dinv_t
        t = jnp.dot(x_ref[...], w1_ref[...],
                    preferred_element_type=jnp.float32)
        tt = t.T * dinv_t
        npad = tt.shape[1]
        cid = jax.lax.broadcasted_iota(jnp.int32, (1, npad), 1)
        tt = jnp.where(cid < n, tt, 0.0)
        z1_out[...] = _split_hi_lo_rows(tt)


def _acc_sym_t(a, zt_j, zt_i, acc_ref, i, j, bm):
    w = zt_j.shape[0]
    ut = jax.lax.dot_general(zt_j, a, (((1,), (1,)), ((), ())),
                             preferred_element_type=jnp.float32)
    acc_ref[:w, pl.ds(i * bm, bm)] += ut

    @pl.when(j > i)
    def _():
        vt = jnp.dot(zt_i, a, preferred_element_type=jnp.float32)
        acc_ref[:w, pl.ds(j * bm, bm)] += vt


def _layers_kernel(a_ref, z1_j, z1_i, dinv_ref, b1_ref, w2t_ref, b2_ref,
                   out_ref, acc_ref, z2_ref, *, bm, ni, f_hid, f_out):
    p = pl.program_id(0)
    i = pl.program_id(1)
    j = pl.program_id(2)

    @pl.when((p == 0) & (i == 0) & (j == 0))
    def _():
        acc_ref[...] = jnp.zeros_like(acc_ref)

    @pl.when(j >= i)
    def _():
        a = a_ref[...].astype(jnp.bfloat16)

        @pl.when(p == 0)
        def _():
            _acc_sym_t(a, z1_j[...], z1_i[...], acc_ref, i, j, bm)

        @pl.when(p == 1)
        def _():
            _acc_sym_t(a, z2_ref[:, pl.ds(j * bm, bm)],
                       z2_ref[:, pl.ds(i * bm, bm)], acc_ref, i, j, bm)

    @pl.when((p == 0) & (i == ni - 1) & (j == ni - 1))
    def _():
        dinv_t = dinv_ref[...]
        acc = acc_ref[...]
        h = (acc[:f_hid, :] + acc[f_hid:, :]) * dinv_t + b1_ref[...]
        h = jnp.maximum(h, 0.0)
        z2t = jnp.dot(w2t_ref[...], h,
                      preferred_element_type=jnp.float32) * dinv_t
        z2_ref[...] = _split_hi_lo_rows(z2t)
        acc_ref[...] = jnp.zeros_like(acc_ref)

    @pl.when((p == 1) & (i == ni - 1) & (j == ni - 1))
    def _():
        acc = acc_ref[...]
        y = (acc[:f_out, :] + acc[f_out:2 * f_out, :]) * dinv_ref[...] \
            + b2_ref[...]
        m = jnp.max(y, axis=0, keepdims=True)
        lse = m + jnp.log(jnp.sum(jnp.exp(y - m), axis=0, keepdims=True))
        out_ref[...] = (y - lse).T


def kernel(x, adj, W1, b1, W2, b2):
    n = adj.shape[0]
    f_in = x.shape[1]
    f_hid = W1.shape[1]
    f_out = W2.shape[1]
    bm = _BM
    bl = _BL
    r = bl // bm
    npad = pl.cdiv(n, bl) * bl
    ni = npad // bm

    def _ja(i, j):
        return jnp.maximum(j, (i // r) * r)

    a_mat, dinv, z1 = pl.pallas_call(
        functools.partial(_sym_deg_kernel, bm=bm, n=n, ni=ni, r=r),
        grid=(ni, ni),
        in_specs=[
            pl.BlockSpec((bm, bm), lambda i, j: (i, _ja(i, j))),
            pl.BlockSpec((bm, bm), lambda i, j: (_ja(i, j), i)),
            pl.BlockSpec((npad, f_in), lambda i, j: (0, 0)),
            pl.BlockSpec((f_in, f_hid), lambda i, j: (0, 0)),
        ],
        out_specs=[
            pl.BlockSpec((bm, bm), lambda i, j: (i, _ja(i, j))),
            pl.BlockSpec((1, npad), lambda i, j: (0, 0)),
            pl.BlockSpec((2 * f_hid, npad), lambda i, j: (0, 0)),
        ],
        out_shape=[
            jax.ShapeDtypeStruct((npad, npad), jnp.int8),
            jax.ShapeDtypeStruct((1, npad), jnp.float32),
            jax.ShapeDtypeStruct((2 * f_hid, npad), jnp.bfloat16),
        ],
        scratch_shapes=[pltpu.VMEM((npad, 1), jnp.float32),
                        pltpu.VMEM((1, npad), jnp.float32)],
        compiler_params=pltpu.CompilerParams(
            dimension_semantics=("arbitrary", "arbitrary")),
    )(adj, adj, x, W1)

    nl = npad // bl

    out = pl.pallas_call(
        functools.partial(_layers_kernel, bm=bl, ni=nl, f_hid=f_hid,
                          f_out=f_out),
        grid=(2, nl, nl),
        in_specs=[
            pl.BlockSpec((bl, bl), lambda p, i, j: (i, jnp.maximum(i, j))),
            pl.BlockSpec((2 * f_hid, bl),
                         lambda p, i, j: (0, jnp.maximum(i, j))),
            pl.BlockSpec((2 * f_hid, bl), lambda p, i, j: (0, i)),
            pl.BlockSpec((1, npad), lambda p, i, j: (0, 0)),
            pl.BlockSpec((f_hid, 1), lambda p, i, j: (0, 0)),
            pl.BlockSpec((f_out, f_hid), lambda p, i, j: (0, 0)),
            pl.BlockSpec((f_out, 1), lambda p, i, j: (0, 0)),
        ],
        out_specs=pl.BlockSpec((npad, f_out), lambda p, i, j: (0, 0)),
        out_shape=jax.ShapeDtypeStruct((npad, f_out), jnp.float32),
        scratch_shapes=[pltpu.VMEM((2 * f_hid, npad), jnp.float32),
                        pltpu.VMEM((2 * f_out, npad), jnp.bfloat16)],
        compiler_params=pltpu.CompilerParams(
            dimension_semantics=("arbitrary", "arbitrary", "arbitrary")),
    )(a_mat, z1, z1, dinv, b1.reshape(f_hid, 1), W2.T, b2.reshape(f_out, 1))

    return out[:n]


# speedup vs baseline: 1.0707x; 1.0707x over previous
import functools

import jax
import jax.numpy as jnp
from jax.experimental import pallas as pl
from jax.experimental.pallas import tpu as pltpu

_BM = 1024
_BL = 2048

_T_DIMS = (((0,), (0,)), ((), ()))


def _split_hi_lo_rows(t):
    hi = t.astype(jnp.bfloat16)
    lo = (t - hi.astype(jnp.float32)).astype(jnp.bfloat16)
    return jnp.concatenate([hi, lo], axis=0)


def _sym_deg_kernel(adj_ij, adj_ji, x_ref, w1_ref, a_out, dinv_out, z1_out,
                    deg_r, deg_c, *, bm, n, ni, r):
    i = pl.program_id(0)
    j = pl.program_id(1)

    @pl.when((i == 0) & (j == 0))
    def _():
        deg_r[...] = jnp.zeros_like(deg_r)
        deg_c[...] = jnp.zeros_like(deg_c)

    def finish(mv):
        a_out[...] = mv.astype(jnp.int8)
        rs = jnp.sum(mv, axis=1, keepdims=True)
        deg_r[pl.ds(i * bm, bm), :] += rs

        @pl.when(j >= (i // r) * r + r)
        def _():
            cs = jnp.sum(mv, axis=0, keepdims=True)
            deg_c[:, pl.ds(j * bm, bm)] += cs

    def sym_and_finish(a, at_src):
        m = jnp.maximum(a, at_src.T)

        is_edge = ((i + 1) * bm > n) | ((j + 1) * bm > n)

        @pl.when(is_edge)
        def _():
            rid = jax.lax.broadcasted_iota(jnp.int32, (bm, 1), 0)
            cid = jax.lax.broadcasted_iota(jnp.int32, (1, bm), 1)
            valid = (rid < n - i * bm) & (cid < n - j * bm)
            finish(jnp.where(valid, m, 0.0))

        @pl.when(~is_edge)
        def _():
            finish(m)

    @pl.when(j == i)
    def _():
        a = adj_ij[...]
        sym_and_finish(a, a)

    @pl.when((j != i) & (j >= (i // r) * r))
    def _():
        sym_and_finish(adj_ij[...], adj_ji[...])

    @pl.when((i == ni - 1) & (j == ni - 1))
    def _():
        deg_t = deg_r[...].T + deg_c[...]
        dinv_t = jnp.where(deg_t > 0.0,
                           jax.lax.rsqrt(jnp.maximum(deg_t, 1e-12)), 0.0)
        dinv_out[...] = dinv_t
        t = jnp.dot(x_ref[...], w1_ref[...],
                    preferred_element_type=jnp.float32)
        tt = t.T * dinv_t
        npad = tt.shape[1]
        cid = jax.lax.broadcasted_iota(jnp.int32, (1, npad), 1)
        tt = jnp.where(cid < n, tt, 0.0)
        z1_out[...] = _split_hi_lo_rows(tt)


def _acc_sym_t(a, zt_j, zt_i, acc_ref, i, j, bm):
    w = zt_j.shape[0]
    ut = jax.lax.dot_general(zt_j, a, (((1,), (1,)), ((), ())),
                             preferred_element_type=jnp.float32)
    acc_ref[:w, pl.ds(i * bm, bm)] += ut

    @pl.when(j > i)
    def _():
        vt = jnp.dot(zt_i, a, preferred_element_type=jnp.float32)
        acc_ref[:w, pl.ds(j * bm, bm)] += vt


def _layers_kernel(a_ref, z1_j, z1_i, dinv_ref, b1_ref, w2t_ref, b2_ref,
                   out_ref, acc_ref, z2_ref, *, bm, ni, f_hid, f_out):
    p = pl.program_id(0)
    i = pl.program_id(1)
    j = pl.program_id(2)

    @pl.when((p == 0) & (i == 0) & (j == 0))
    def _():
        acc_ref[...] = jnp.zeros_like(acc_ref)

    @pl.when(j >= i)
    def _():
        a = a_ref[...].astype(jnp.bfloat16)

        @pl.when(p == 0)
        def _():
            _acc_sym_t(a, z1_j[...], z1_i[...], acc_ref, i, j, bm)

        @pl.when(p == 1)
        def _():
            _acc_sym_t(a, z2_ref[:, pl.ds(j * bm, bm)],
                       z2_ref[:, pl.ds(i * bm, bm)], acc_ref, i, j, bm)

    @pl.when((p == 0) & (i == ni - 1) & (j == ni - 1))
    def _():
        dinv_t = dinv_ref[...]
        acc = acc_ref[...]
        h = (acc[:f_hid, :] + acc[f_hid:, :]) * dinv_t + b1_ref[...]
        h = jnp.maximum(h, 0.0)
        z2t = jnp.dot(w2t_ref[...], h,
                      preferred_element_type=jnp.float32) * dinv_t
        z2_ref[...] = _split_hi_lo_rows(z2t)
        acc_ref[...] = jnp.zeros_like(acc_ref)

    @pl.when((p == 1) & (i == ni - 1) & (j == ni - 1))
    def _():
        acc = acc_ref[...]
        y = (acc[:f_out, :] + acc[f_out:2 * f_out, :]) * dinv_ref[...] \
            + b2_ref[...]
        m = jnp.max(y, axis=0, keepdims=True)
        lse = m + jnp.log(jnp.sum(jnp.exp(y - m), axis=0, keepdims=True))
        out_ref[...] = (y - lse).T


def kernel(x, adj, W1, b1, W2, b2):
    n = adj.shape[0]
    f_in = x.shape[1]
    f_hid = W1.shape[1]
    f_out = W2.shape[1]
    bm = _BM
    bl = _BL
    r = bl // bm
    npad = pl.cdiv(n, bl) * bl
    ni = npad // bm

    def _ja(i, j):
        return jnp.maximum(j, (i // r) * r)

    def _jb(i, j):
        jj = _ja(i, j)
        return jnp.where(jj == i, jnp.minimum(i + 1, ni - 1), jj)

    a_mat, dinv, z1 = pl.pallas_call(
        functools.partial(_sym_deg_kernel, bm=bm, n=n, ni=ni, r=r),
        grid=(ni, ni),
        in_specs=[
            pl.BlockSpec((bm, bm), lambda i, j: (i, _ja(i, j))),
            pl.BlockSpec((bm, bm), lambda i, j: (_jb(i, j), i)),
            pl.BlockSpec((npad, f_in), lambda i, j: (0, 0)),
            pl.BlockSpec((f_in, f_hid), lambda i, j: (0, 0)),
        ],
        out_specs=[
            pl.BlockSpec((bm, bm), lambda i, j: (i, _ja(i, j))),
            pl.BlockSpec((1, npad), lambda i, j: (0, 0)),
            pl.BlockSpec((2 * f_hid, npad), lambda i, j: (0, 0)),
        ],
        out_shape=[
            jax.ShapeDtypeStruct((npad, npad), jnp.int8),
            jax.ShapeDtypeStruct((1, npad), jnp.float32),
            jax.ShapeDtypeStruct((2 * f_hid, npad), jnp.bfloat16),
        ],
        scratch_shapes=[pltpu.VMEM((npad, 1), jnp.float32),
                        pltpu.VMEM((1, npad), jnp.float32)],
        compiler_params=pltpu.CompilerParams(
            dimension_semantics=("arbitrary", "arbitrary")),
    )(adj, adj, x, W1)

    nl = npad // bl

    out = pl.pallas_call(
        functools.partial(_layers_kernel, bm=bl, ni=nl, f_hid=f_hid,
                          f_out=f_out),
        grid=(2, nl, nl),
        in_specs=[
            pl.BlockSpec((bl, bl), lambda p, i, j: (i, jnp.maximum(i, j))),
            pl.BlockSpec((2 * f_hid, bl),
                         lambda p, i, j: (0, jnp.maximum(i, j))),
            pl.BlockSpec((2 * f_hid, bl), lambda p, i, j: (0, i)),
            pl.BlockSpec((1, npad), lambda p, i, j: (0, 0)),
            pl.BlockSpec((f_hid, 1), lambda p, i, j: (0, 0)),
            pl.BlockSpec((f_out, f_hid), lambda p, i, j: (0, 0)),
            pl.BlockSpec((f_out, 1), lambda p, i, j: (0, 0)),
        ],
        out_specs=pl.BlockSpec((npad, f_out), lambda p, i, j: (0, 0)),
        out_shape=jax.ShapeDtypeStruct((npad, f_out), jnp.float32),
        scratch_shapes=[pltpu.VMEM((2 * f_hid, npad), jnp.float32),
                        pltpu.VMEM((2 * f_out, npad), jnp.bfloat16)],
        compiler_params=pltpu.CompilerParams(
            dimension_semantics=("arbitrary", "arbitrary", "arbitrary")),
    )(a_mat, z1, z1, dinv, b1.reshape(f_hid, 1), W2.T, b2.reshape(f_out, 1))

    return out[:n]


# speedup vs baseline: 1.0986x; 1.0261x over previous
import functools

import jax
import jax.numpy as jnp
from jax.experimental import pallas as pl
from jax.experimental.pallas import tpu as pltpu

_BM = 1024
_BL = 2048

_T_DIMS = (((0,), (0,)), ((), ()))


def _split_hi_lo_rows(t):
    hi = t.astype(jnp.bfloat16)
    lo = (t - hi.astype(jnp.float32)).astype(jnp.bfloat16)
    return jnp.concatenate([hi, lo], axis=0)


def _tri_starts(ni, r):
    starts = []
    s = 0
    for i in range(ni):
        starts.append(s)
        s += ni - (i // r) * r
    return tuple(starts), s


def _tri_ij(p, starts, r):
    i = jnp.int32(0)
    for k in range(1, len(starts)):
        i = jnp.where(p >= starts[k], jnp.int32(k), i)
    st = jnp.int32(0)
    for k in range(len(starts)):
        st = jnp.where(i == k, jnp.int32(starts[k]), st)
    j = p - st + (i // r) * r
    return i, j


def _sym_deg_kernel(adj_ij, adj_ji, x_ref, w1_ref, a_out, dinv_out, z1_out,
                    deg_r, deg_c, *, bm, n, ni, r, starts, tot):
    p = pl.program_id(0)
    i, j = _tri_ij(p, starts, r)

    @pl.when(p == 0)
    def _():
        deg_r[...] = jnp.zeros_like(deg_r)
        deg_c[...] = jnp.zeros_like(deg_c)

    def finish(mv):
        a_out[...] = mv.astype(jnp.int8)
        rs = jnp.sum(mv, axis=1, keepdims=True)
        deg_r[pl.ds(i * bm, bm), :] += rs

        @pl.when(j >= (i // r) * r + r)
        def _():
            cs = jnp.sum(mv, axis=0, keepdims=True)
            deg_c[:, pl.ds(j * bm, bm)] += cs

    def sym_and_finish(a, at_src):
        m = jnp.maximum(a, at_src.T)

        is_edge = ((i + 1) * bm > n) | ((j + 1) * bm > n)

        @pl.when(is_edge)
        def _():
            rid = jax.lax.broadcasted_iota(jnp.int32, (bm, 1), 0)
            cid = jax.lax.broadcasted_iota(jnp.int32, (1, bm), 1)
            valid = (rid < n - i * bm) & (cid < n - j * bm)
            finish(jnp.where(valid, m, 0.0))

        @pl.when(~is_edge)
        def _():
            finish(m)

    @pl.when(j == i)
    def _():
        a = adj_ij[...]
        sym_and_finish(a, a)

    @pl.when(j != i)
    def _():
        sym_and_finish(adj_ij[...], adj_ji[...])

    @pl.when(p == tot - 1)
    def _():
        deg_t = deg_r[...].T + deg_c[...]
        dinv_t = jnp.where(deg_t > 0.0,
                           jax.lax.rsqrt(jnp.maximum(deg_t, 1e-12)), 0.0)
        dinv_out[...] = dinv_t
        t = jnp.dot(x_ref[...], w1_ref[...],
                    preferred_element_type=jnp.float32)
        tt = t.T * dinv_t
        npad = tt.shape[1]
        cid = jax.lax.broadcasted_iota(jnp.int32, (1, npad), 1)
        tt = jnp.where(cid < n, tt, 0.0)
        z1_out[...] = _split_hi_lo_rows(tt)


def _acc_sym_t(a, zt_j, zt_i, acc_ref, i, j, bm):
    w = zt_j.shape[0]
    ut = jax.lax.dot_general(zt_j, a, (((1,), (1,)), ((), ())),
                             preferred_element_type=jnp.float32)
    acc_ref[:w, pl.ds(i * bm, bm)] += ut

    @pl.when(j > i)
    def _():
        vt = jnp.dot(zt_i, a, preferred_element_type=jnp.float32)
        acc_ref[:w, pl.ds(j * bm, bm)] += vt


def _layers_kernel(a_ref, z1_j, z1_i, dinv_ref, b1_ref, w2t_ref, b2_ref,
                   out_ref, acc_ref, z2_ref, *, bm, ni, f_hid, f_out,
                   starts, tot):
    p = pl.program_id(0)
    q = pl.program_id(1)
    i, j = _tri_ij(q, starts, 1)

    @pl.when((p == 0) & (q == 0))
    def _():
        acc_ref[...] = jnp.zeros_like(acc_ref)

    a = a_ref[...].astype(jnp.bfloat16)

    @pl.when(p == 0)
    def _():
        _acc_sym_t(a, z1_j[...], z1_i[...], acc_ref, i, j, bm)

    @pl.when(p == 1)
    def _():
        _acc_sym_t(a, z2_ref[:, pl.ds(j * bm, bm)],
                   z2_ref[:, pl.ds(i * bm, bm)], acc_ref, i, j, bm)

    @pl.when((p == 0) & (q == tot - 1))
    def _():
        dinv_t = dinv_ref[...]
        acc = acc_ref[...]
        h = (acc[:f_hid, :] + acc[f_hid:, :]) * dinv_t + b1_ref[...]
        h = jnp.maximum(h, 0.0)
        z2t = jnp.dot(w2t_ref[...], h,
                      preferred_element_type=jnp.float32) * dinv_t
        z2_ref[...] = _split_hi_lo_rows(z2t)
        acc_ref[...] = jnp.zeros_like(acc_ref)

    @pl.when((p == 1) & (q == tot - 1))
    def _():
        acc = acc_ref[...]
        y = (acc[:f_out, :] + acc[f_out:2 * f_out, :]) * dinv_ref[...] \
            + b2_ref[...]
        m = jnp.max(y, axis=0, keepdims=True)
        lse = m + jnp.log(jnp.sum(jnp.exp(y - m), axis=0, keepdims=True))
        out_ref[...] = (y - lse).T


def kernel(x, adj, W1, b1, W2, b2):
    n = adj.shape[0]
    f_in = x.shape[1]
    f_hid = W1.shape[1]
    f_out = W2.shape[1]
    bm = _BM
    bl = _BL
    r = bl // bm
    npad = pl.cdiv(n, bl) * bl
    ni = npad // bm

    starts0, tot0 = _tri_starts(ni, r)

    def _im_a(p):
        return _tri_ij(p, starts0, r)

    def _im_at(p):
        i, j = _tri_ij(p, starts0, r)
        return (jnp.where(j == i, jnp.minimum(i + 1, ni - 1), j), i)

    a_mat, dinv, z1 = pl.pallas_call(
        functools.partial(_sym_deg_kernel, bm=bm, n=n, ni=ni, r=r,
                          starts=starts0, tot=tot0),
        grid=(tot0,),
        in_specs=[
            pl.BlockSpec((bm, bm), _im_a),
            pl.BlockSpec((bm, bm), _im_at),
            pl.BlockSpec((npad, f_in), lambda p: (0, 0)),
            pl.BlockSpec((f_in, f_hid), lambda p: (0, 0)),
        ],
        out_specs=[
            pl.BlockSpec((bm, bm), _im_a),
            pl.BlockSpec((1, npad), lambda p: (0, 0)),
            pl.BlockSpec((2 * f_hid, npad), lambda p: (0, 0)),
        ],
        out_shape=[
            jax.ShapeDtypeStruct((npad, npad), jnp.int8),
            jax.ShapeDtypeStruct((1, npad), jnp.float32),
            jax.ShapeDtypeStruct((2 * f_hid, npad), jnp.bfloat16),
        ],
        scratch_shapes=[pltpu.VMEM((npad, 1), jnp.float32),
                        pltpu.VMEM((1, npad), jnp.float32)],
        compiler_params=pltpu.CompilerParams(
            dimension_semantics=("arbitrary",)),
    )(adj, adj, x, W1)

    nl = npad // bl
    startsl, totl = _tri_starts(nl, 1)

    def _lm_a(p, q):
        return _tri_ij(q, startsl, 1)

    def _lm_zj(p, q):
        return (0, _tri_ij(q, startsl, 1)[1])

    def _lm_zi(p, q):
        return (0, _tri_ij(q, startsl, 1)[0])

    out = pl.pallas_call(
        functools.partial(_layers_kernel, bm=bl, ni=nl, f_hid=f_hid,
                          f_out=f_out, starts=startsl, tot=totl),
        grid=(2, totl),
        in_specs=[
            pl.BlockSpec((bl, bl), _lm_a),
            pl.BlockSpec((2 * f_hid, bl), _lm_zj),
            pl.BlockSpec((2 * f_hid, bl), _lm_zi),
            pl.BlockSpec((1, npad), lambda p, q: (0, 0)),
            pl.BlockSpec((f_hid, 1), lambda p, q: (0, 0)),
            pl.BlockSpec((f_out, f_hid), lambda p, q: (0, 0)),
            pl.BlockSpec((f_out, 1), lambda p, q: (0, 0)),
        ],
        out_specs=pl.BlockSpec((npad, f_out), lambda p, q: (0, 0)),
        out_shape=jax.ShapeDtypeStruct((npad, f_out), jnp.float32),
        scratch_shapes=[pltpu.VMEM((2 * f_hid, npad), jnp.float32),
                        pltpu.VMEM((2 * f_out, npad), jnp.bfloat16)],
        compiler_params=pltpu.CompilerParams(
            dimension_semantics=("arbitrary", "arbitrary")),
    )(a_mat, z1, z1, dinv, b1.reshape(f_hid, 1), W2.T, b2.reshape(f_out, 1))

    return out[:n]


# speedup vs baseline: 1.1004x; 1.0017x over previous
import functools

import jax
import jax.numpy as jnp
from jax.experimental import pallas as pl
from jax.experimental.pallas import tpu as pltpu

_BM = 1024
_BL = 2048

def _split_hi_lo_rows(t):
    hi = t.astype(jnp.bfloat16)
    lo = (t - hi.astype(jnp.float32)).astype(jnp.bfloat16)
    return jnp.concatenate([hi, lo], axis=0)


def _tri_starts(ni, r):
    starts = []
    s = 0
    for i in range(ni):
        starts.append(s)
        s += ni - (i // r) * r
    return tuple(starts), s


def _tri_ij(p, starts, r):
    i = jnp.int32(0)
    for k in range(1, len(starts)):
        i = jnp.where(p >= starts[k], jnp.int32(k), i)
    st = jnp.int32(0)
    for k in range(len(starts)):
        st = jnp.where(i == k, jnp.int32(starts[k]), st)
    j = p - st + (i // r) * r
    return i, j


def _sym_deg_kernel(adj_ij, adj_ji, x_ref, w1_ref, a_out, dinv_out, z1_out,
                    deg_r, deg_c, *, bm, n, ni, r, starts, tot):
    p = pl.program_id(0)
    i, j = _tri_ij(p, starts, r)

    @pl.when(p == 0)
    def _():
        deg_r[...] = jnp.zeros_like(deg_r)
        deg_c[...] = jnp.zeros_like(deg_c)

    def finish(mv):
        a_out[...] = mv.astype(jnp.int8)
        rs = jnp.sum(mv, axis=1, keepdims=True)
        deg_r[pl.ds(i * bm, bm), :] += rs

        @pl.when(j >= (i // r) * r + r)
        def _():
            cs = jnp.sum(mv, axis=0, keepdims=True)
            deg_c[:, pl.ds(j * bm, bm)] += cs

    def sym_and_finish(a, at_src):
        m = jnp.maximum(a, at_src.T)

        is_edge = ((i + 1) * bm > n) | ((j + 1) * bm > n)

        @pl.when(is_edge)
        def _():
            rid = jax.lax.broadcasted_iota(jnp.int32, (bm, 1), 0)
            cid = jax.lax.broadcasted_iota(jnp.int32, (1, bm), 1)
            valid = (rid < n - i * bm) & (cid < n - j * bm)
            finish(jnp.where(valid, m, 0.0))

        @pl.when(~is_edge)
        def _():
            finish(m)

    @pl.when(j == i)
    def _():
        a = adj_ij[...]
        sym_and_finish(a, a)

    @pl.when(j != i)
    def _():
        sym_and_finish(adj_ij[...], adj_ji[...])

    @pl.when(p == tot - 1)
    def _():
        deg_t = deg_r[...].T + deg_c[...]
        dinv_t = jnp.where(deg_t > 0.0,
                           jax.lax.rsqrt(jnp.maximum(deg_t, 1e-12)), 0.0)
        dinv_out[...] = dinv_t
        t = jnp.dot(x_ref[...], w1_ref[...],
                    preferred_element_type=jnp.float32)
        tt = t.T * dinv_t
        npad = tt.shape[1]
        cid = jax.lax.broadcasted_iota(jnp.int32, (1, npad), 1)
        tt = jnp.where(cid < n, tt, 0.0)
        z1_out[...] = _split_hi_lo_rows(tt)


def _acc_sym_t(a, zt_j, zt_i, acc_ref, i, j, bm):
    w = zt_j.shape[0]
    ut = jax.lax.dot_general(zt_j, a, (((1,), (1,)), ((), ())),
                             preferred_element_type=jnp.float32)
    acc_ref[:w, pl.ds(i * bm, bm)] += ut

    @pl.when(j > i)
    def _():
        vt = jnp.dot(zt_i, a, preferred_element_type=jnp.float32)
        acc_ref[:w, pl.ds(j * bm, bm)] += vt


def _layers_kernel(a_ref, z1_j, z1_i, dinv_ref, b1_ref, w2t_ref, b2_ref,
                   out_ref, acc_ref, z2_ref, *, bm, ni, f_hid, f_out,
                   starts, tot):
    p = pl.program_id(0)
    q = pl.program_id(1)
    i, j = _tri_ij(q, starts, 1)

    @pl.when((p == 0) & (q == 0))
    def _():
        acc_ref[...] = jnp.zeros_like(acc_ref)

    a = a_ref[...].astype(jnp.bfloat16)

    @pl.when(p == 0)
    def _():
        _acc_sym_t(a, z1_j[...], z1_i[...], acc_ref, i, j, bm)

    @pl.when(p == 1)
    def _():
        _acc_sym_t(a, z2_ref[:, pl.ds(j * bm, bm)],
                   z2_ref[:, pl.ds(i * bm, bm)], acc_ref, i, j, bm)

    @pl.when((p == 0) & (q == tot - 1))
    def _():
        dinv_t = dinv_ref[...]
        acc = acc_ref[...]
        h = (acc[:f_hid, :] + acc[f_hid:, :]) * dinv_t + b1_ref[...]
        h = jnp.maximum(h, 0.0)
        z2t = jnp.dot(w2t_ref[...], h,
                      preferred_element_type=jnp.float32) * dinv_t
        z2_ref[...] = _split_hi_lo_rows(z2t)
        acc_ref[...] = jnp.zeros_like(acc_ref)

    @pl.when((p == 1) & (q == tot - 1))
    def _():
        acc = acc_ref[...]
        y = (acc[:f_out, :] + acc[f_out:2 * f_out, :]) * dinv_ref[...] \
            + b2_ref[...]
        m = jnp.max(y, axis=0, keepdims=True)
        lse = m + jnp.log(jnp.sum(jnp.exp(y - m), axis=0, keepdims=True))
        out_ref[...] = (y - lse).T


def kernel(x, adj, W1, b1, W2, b2):
    n = adj.shape[0]
    f_in = x.shape[1]
    f_hid = W1.shape[1]
    f_out = W2.shape[1]
    bm = _BM
    bl = _BL
    r = bl // bm
    npad = pl.cdiv(n, bl) * bl
    ni = npad // bm

    starts0, tot0 = _tri_starts(ni, r)

    def _im_a(p):
        return _tri_ij(p, starts0, r)

    def _im_at(p):
        i, j = _tri_ij(p, starts0, r)
        return (jnp.where(j == i, jnp.minimum(i + 1, ni - 1), j), i)

    a_mat, dinv, z1 = pl.pallas_call(
        functools.partial(_sym_deg_kernel, bm=bm, n=n, ni=ni, r=r,
                          starts=starts0, tot=tot0),
        grid=(tot0,),
        in_specs=[
            pl.BlockSpec((bm, bm), _im_a),
            pl.BlockSpec((bm, bm), _im_at),
            pl.BlockSpec((npad, f_in), lambda p: (0, 0)),
            pl.BlockSpec((f_in, f_hid), lambda p: (0, 0)),
        ],
        out_specs=[
            pl.BlockSpec((bm, bm), _im_a),
            pl.BlockSpec((1, npad), lambda p: (0, 0)),
            pl.BlockSpec((2 * f_hid, npad), lambda p: (0, 0)),
        ],
        out_shape=[
            jax.ShapeDtypeStruct((npad, npad), jnp.int8),
            jax.ShapeDtypeStruct((1, npad), jnp.float32),
            jax.ShapeDtypeStruct((2 * f_hid, npad), jnp.bfloat16),
        ],
        scratch_shapes=[pltpu.VMEM((npad, 1), jnp.float32),
                        pltpu.VMEM((1, npad), jnp.float32)],
        compiler_params=pltpu.CompilerParams(
            dimension_semantics=("arbitrary",)),
    )(adj, adj, x, W1)

    nl = npad // bl
    startsl, totl = _tri_starts(nl, 1)

    def _lm_a(p, q):
        return _tri_ij(q, startsl, 1)

    def _lm_zj(p, q):
        return (0, _tri_ij(q, startsl, 1)[1])

    def _lm_zi(p, q):
        return (0, _tri_ij(q, startsl, 1)[0])

    out = pl.pallas_call(
        functools.partial(_layers_kernel, bm=bl, ni=nl, f_hid=f_hid,
                          f_out=f_out, starts=startsl, tot=totl),
        grid=(2, totl),
        in_specs=[
            pl.BlockSpec((bl, bl), _lm_a),
            pl.BlockSpec((2 * f_hid, bl), _lm_zj),
            pl.BlockSpec((2 * f_hid, bl), _lm_zi),
            pl.BlockSpec((1, npad), lambda p, q: (0, 0)),
            pl.BlockSpec((f_hid, 1), lambda p, q: (0, 0)),
            pl.BlockSpec((f_out, f_hid), lambda p, q: (0, 0)),
            pl.BlockSpec((f_out, 1), lambda p, q: (0, 0)),
        ],
        out_specs=pl.BlockSpec((npad, f_out), lambda p, q: (0, 0)),
        out_shape=jax.ShapeDtypeStruct((npad, f_out), jnp.float32),
        scratch_shapes=[pltpu.VMEM((2 * f_hid, npad), jnp.float32),
                        pltpu.VMEM((2 * f_out, npad), jnp.bfloat16)],
        compiler_params=pltpu.CompilerParams(
            dimension_semantics=("arbitrary", "arbitrary")),
    )(a_mat, z1, z1, dinv, b1.reshape(f_hid, 1), W2.T, b2.reshape(f_out, 1))

    return out[:n]
